# baseline (speedup 1.0000x reference)
import functools

import jax
import jax.numpy as jnp
from jax import lax
from jax.experimental import pallas as pl
from jax.experimental.pallas import tpu as pltpu

C = 768; T = 24576; SRC = 128; D = 256; L = 96; H = 4; DH = 64; Q = 4; FF = 1024
S = C * Q
CB = 4
N = CB * L
NSEG = 12
TB = T // NSEG
QB = 768
SCALE = 0.125
NEG = -1e9


def _seg_kernel(seg_ref, starts_ref, lens_ref, acc_lt, acc_eq):
    i = pl.program_id(0)

    @pl.when(i == 0)
    def _init():
        acc_lt[...] = jnp.zeros_like(acc_lt)
        acc_eq[...] = jnp.zeros_like(acc_eq)

    seg = seg_ref[0]
    cells = lax.broadcasted_iota(jnp.int32, (1, C), 1)
    lt = (seg < cells).astype(jnp.float32)
    eq = (seg == cells).astype(jnp.float32)
    acc_lt[...] += jnp.sum(lt, axis=0, keepdims=True)
    acc_eq[...] += jnp.sum(eq, axis=0, keepdims=True)

    @pl.when(i == NSEG - 1)
    def _fin():
        starts_ref[...] = acc_lt[...].astype(jnp.int32)
        lens_ref[...] = acc_eq[...].astype(jnp.int32)


def _mm(a, b):
    return lax.dot_general(a.astype(jnp.bfloat16), b.astype(jnp.bfloat16),
                           (((1,), (0,)), ((), ())),
                           preferred_element_type=jnp.float32)


def _mmt(a, b):
    return lax.dot_general(a.astype(jnp.bfloat16), b.astype(jnp.bfloat16),
                           (((1,), (1,)), ((), ())),
                           preferred_element_type=jnp.float32)


def _masked_attn(qh, kh, vh, allow, allowf):
    s = _mmt(qh, kh) * SCALE
    s = jnp.where(allow, s, NEG)
    m = jnp.max(s, axis=1, keepdims=True)
    p = jnp.exp(s - m) * allowf
    den = jnp.maximum(jnp.sum(p, axis=1, keepdims=True), 1e-30)
    return _mm(p, vh) / den


def _local_kernel(starts, lens, streams, qc, pe, we, be, wq, wk, wv, wo,
                  w1, b1, w2, b2, wq2, wk2, wv2, wo2, out, xbuf, sem):
    pid = pl.program_id(0)
    c0 = pid * CB
    shifts, clens, copies = [], [], []
    for j in range(CB):
        st = starts[c0 + j]
        cl = jnp.minimum(lens[c0 + j], L)
        s0 = jnp.minimum(st, T - L)
        shifts.append(st - s0)
        clens.append(cl)
        cp = pltpu.make_async_copy(streams.at[pl.ds(s0, L), :], xbuf.at[j],
                                   sem.at[j])
        cp.start()
        copies.append(cp)

    rcol = lax.broadcasted_iota(jnp.int32, (N, 1), 0)
    rrow = lax.broadcasted_iota(jnp.int32, (1, N), 1)

    def valid_of(r):
        cidx = r // L
        rl = r % L
        sh = jnp.zeros_like(r)
        cl = jnp.zeros_like(r)
        for j in range(CB):
            sh = jnp.where(cidx == j, shifts[j], sh)
            cl = jnp.where(cidx == j, clens[j], cl)
        return (rl >= sh) & (rl < sh + cl)

    vcol = valid_of(rcol)
    vrow = valid_of(rrow)
    maskf = vcol.astype(jnp.float32)

    for cp in copies:
        cp.wait()

    x = _mm(xbuf[...].reshape(N, SRC), we[...]) + be[...]
    x = x * maskf

    q = _mm(x, wq[...]); k = _mm(x, wk[...]); v = _mm(x, wv[...])
    ri = lax.broadcasted_iota(jnp.int32, (N, N), 0) // L
    ci = lax.broadcasted_iota(jnp.int32, (N, N), 1) // L
    allow = (ri == ci) & vrow
    allowf = allow.astype(jnp.float32)
    os_ = [_masked_attn(q[:, DH*h:DH*(h+1)], k[:, DH*h:DH*(h+1)],
                        v[:, DH*h:DH*(h+1)], allow, allowf) for h in range(H)]
    o = _mm(jnp.concatenate(os_, axis=1), wo[...])
    h1 = x + o * maskf
    h2 = h1 + (_mm(jax.nn.gelu(_mm(h1, w1[...]) + b1[...]), w2[...])
               + b2[...]) * maskf

    k2 = _mm(h2, wk2[...]); v2 = _mm(h2, wv2[...])
    qpe = (qc[...] + pe[...]).reshape(CB * Q, D)
    qg = _mm(qpe, wq2[...])
    ri2 = lax.broadcasted_iota(jnp.int32, (CB * Q, N), 0) // Q
    ci2 = lax.broadcasted_iota(jnp.int32, (CB * Q, N), 1) // L
    allow2 = (ri2 == ci2) & vrow
    allow2f = allow2.astype(jnp.float32)
    gs = [_masked_attn(qg[:, DH*h:DH*(h+1)], k2[:, DH*h:DH*(h+1)],
                       v2[:, DH*h:DH*(h+1)], allow2, allow2f) for h in range(H)]
    g = _mm(jnp.concatenate(gs, axis=1), wo2[...])
    out[...] = (qpe + pe[...].reshape(CB * Q, D) + g).reshape(CB, Q, D)


def _kv_kernel(tg_ref, wkg, wvg, kk_ref, vv_ref):
    t = tg_ref[...]
    kk_ref[...] = _mm(t, wkg[...])
    vv_ref[...] = _mm(t, wvg[...])


def _global_kernel(tg_ref, kk_ref, vv_ref, wqg, wog, wg1, bg1, wg2, bg2, z_ref):
    t = tg_ref[...]
    qq = _mm(t, wqg[...])
    kk = kk_ref[...]
    vv = vv_ref[...]
    outs = []
    for h in range(H):
        s = _mmt(qq[:, DH*h:DH*(h+1)], kk[:, DH*h:DH*(h+1)]) * SCALE
        m = jnp.max(s, axis=1, keepdims=True)
        p = jnp.exp(s - m)
        a = p / jnp.sum(p, axis=1, keepdims=True)
        outs.append(_mm(a, vv[:, DH*h:DH*(h+1)]))
    z = t + _mm(jnp.concatenate(outs, axis=1), wog[...])
    z = z + _mm(jax.nn.gelu(_mm(z, wg1[...]) + bg1[...]), wg2[...]) + bg2[...]
    z_ref[...] = z


def _full(shape):
    return pl.BlockSpec(shape, lambda i, *_: (0,) * len(shape))


def kernel(streams_data, segment_ids, pe_global, q_cells, W_embed, b_embed,
           Wq, Wk, Wv, Wo, W1, b1, W2, b2, Wq2, Wk2, Wv2, Wo2,
           Wqg, Wkg, Wvg, Wog, Wg1, bg1, Wg2, bg2):
    starts2, lens2 = pl.pallas_call(
        _seg_kernel,
        grid=(NSEG,),
        in_specs=[pl.BlockSpec((1, TB, 1), lambda i: (i, 0, 0))],
        out_specs=[pl.BlockSpec((1, C), lambda i: (0, 0))] * 2,
        out_shape=[jax.ShapeDtypeStruct((1, C), jnp.int32)] * 2,
        scratch_shapes=[pltpu.VMEM((1, C), jnp.float32)] * 2,
    )(segment_ids.reshape(NSEG, TB, 1))
    starts = starts2.reshape(C)
    lens = lens2.reshape(C)

    grid_spec = pltpu.PrefetchScalarGridSpec(
        num_scalar_prefetch=2,
        grid=(C // CB,),
        in_specs=[
            pl.BlockSpec(memory_space=pl.ANY),
            pl.BlockSpec((CB, Q, D), lambda i, *_: (i, 0, 0)),
            pl.BlockSpec((CB, Q, D), lambda i, *_: (i, 0, 0)),
            _full((SRC, D)), _full((1, D)),
            _full((D, D)), _full((D, D)), _full((D, D)), _full((D, D)),
            _full((D, FF)), _full((1, FF)), _full((FF, D)), _full((1, D)),
            _full((D, D)), _full((D, D)), _full((D, D)), _full((D, D)),
        ],
        out_specs=pl.BlockSpec((CB, Q, D), lambda i, *_: (i, 0, 0)),
        scratch_shapes=[pltpu.VMEM((CB, L, SRC), jnp.float32),
                        pltpu.SemaphoreType.DMA((CB,))],
    )
    tg = pl.pallas_call(
        _local_kernel,
        grid_spec=grid_spec,
        out_shape=jax.ShapeDtypeStruct((C, Q, D), jnp.float32),
    )(starts, lens, streams_data, q_cells, pe_global,
      W_embed, b_embed.reshape(1, D), Wq, Wk, Wv, Wo,
      W1, b1.reshape(1, FF), W2, b2.reshape(1, D), Wq2, Wk2, Wv2, Wo2)

    tg2 = tg.reshape(S, D)
    kk, vv = pl.pallas_call(
        _kv_kernel,
        grid=(S // QB,),
        in_specs=[pl.BlockSpec((QB, D), lambda i: (i, 0)),
                  pl.BlockSpec((D, D), lambda i: (0, 0)),
                  pl.BlockSpec((D, D), lambda i: (0, 0))],
        out_specs=[pl.BlockSpec((QB, D), lambda i: (i, 0))] * 2,
        out_shape=[jax.ShapeDtypeStruct((S, D), jnp.float32)] * 2,
    )(tg2, Wkg, Wvg)

    z = pl.pallas_call(
        _global_kernel,
        grid=(S // QB,),
        in_specs=[pl.BlockSpec((QB, D), lambda i: (i, 0)),
                  pl.BlockSpec((S, D), lambda i: (0, 0)),
                  pl.BlockSpec((S, D), lambda i: (0, 0)),
                  pl.BlockSpec((D, D), lambda i: (0, 0)),
                  pl.BlockSpec((D, D), lambda i: (0, 0)),
                  pl.BlockSpec((D, FF), lambda i: (0, 0)),
                  pl.BlockSpec((1, FF), lambda i: (0, 0)),
                  pl.BlockSpec((FF, D), lambda i: (0, 0)),
                  pl.BlockSpec((1, D), lambda i: (0, 0))],
        out_specs=pl.BlockSpec((QB, D), lambda i: (i, 0)),
        out_shape=jax.ShapeDtypeStruct((S, D), jnp.float32),
    )(tg2, kk, vv, Wqg, Wog, Wg1, bg1.reshape(1, FF), Wg2, bg2.reshape(1, D))
    return z


# speedup vs baseline: 1.2581x; 1.2581x over previous
import functools

import jax
import jax.numpy as jnp
import numpy as np
from jax import lax
from jax.experimental import pallas as pl
from jax.experimental.pallas import tpu as pltpu

C = 768; T = 24576; SRC = 128; D = 256; L = 96; H = 4; DH = 64; Q = 4; FF = 1024
S = C * Q
CB = 4
N = CB * L
NSEG = 12
TB = T // NSEG
QB = 768
SCALE = 0.125
NEG = -1e9


def _seg_kernel(seg_ref, starts_ref, lens_ref, acc_lt, acc_eq):
    i = pl.program_id(0)

    @pl.when(i == 0)
    def _init():
        acc_lt[...] = jnp.zeros_like(acc_lt)
        acc_eq[...] = jnp.zeros_like(acc_eq)

    seg = seg_ref[0]
    cells = lax.broadcasted_iota(jnp.int32, (1, C), 1)
    lt = (seg < cells).astype(jnp.float32)
    eq = (seg == cells).astype(jnp.float32)
    acc_lt[...] += jnp.sum(lt, axis=0, keepdims=True)
    acc_eq[...] += jnp.sum(eq, axis=0, keepdims=True)

    @pl.when(i == NSEG - 1)
    def _fin():
        starts_ref[...] = acc_lt[...].astype(jnp.int32)
        lens_ref[...] = acc_eq[...].astype(jnp.int32)


def _mm(a, b):
    return lax.dot_general(a.astype(jnp.bfloat16), b.astype(jnp.bfloat16),
                           (((1,), (0,)), ((), ())),
                           preferred_element_type=jnp.float32)


def _mmt(a, b):
    return lax.dot_general(a.astype(jnp.bfloat16), b.astype(jnp.bfloat16),
                           (((1,), (1,)), ((), ())),
                           preferred_element_type=jnp.float32)


def _masked_attn(qh, kh, vh, allow, allowf):
    s = _mmt(qh, kh) * SCALE
    s = jnp.where(allow, s, NEG)
    m = jnp.max(s, axis=1, keepdims=True)
    p = jnp.exp(s - m) * allowf
    den = jnp.maximum(jnp.sum(p, axis=1, keepdims=True), 1e-30)
    return _mm(p, vh) / den


def _issue_copies(starts, lens, streams, xbuf, sem, step, slot):
    c0 = step * CB
    shifts, clens, copies = [], [], []
    for j in range(CB):
        st = starts[c0 + j]
        cl = jnp.minimum(lens[c0 + j], L)
        s0 = jnp.minimum(st, T - L)
        shifts.append(st - s0)
        clens.append(cl)
        copies.append(pltpu.make_async_copy(
            streams.at[pl.ds(s0, L), :], xbuf.at[slot, j], sem.at[slot, j]))
    return shifts, clens, copies


def _local_kernel(starts, lens, streams, qc, pe, bias, we, be, wq, wk, wv, wo,
                  w1, b1, w2, b2, wq2, wk2, wv2, wo2, out, xbuf, sem):
    pid = pl.program_id(0)
    G = pl.num_programs(0)
    slot = lax.rem(pid, 2)

    @pl.when(pid == 0)
    def _prime():
        for cp in _issue_copies(starts, lens, streams, xbuf, sem, 0, 0)[2]:
            cp.start()

    @pl.when(pid < G - 1)
    def _next():
        for cp in _issue_copies(starts, lens, streams, xbuf, sem,
                                pid + 1, lax.rem(pid + 1, 2))[2]:
            cp.start()

    shifts, clens, copies = _issue_copies(starts, lens, streams, xbuf, sem,
                                          pid, slot)

    rcol = lax.broadcasted_iota(jnp.int32, (N, 1), 0)
    rrow = lax.broadcasted_iota(jnp.int32, (1, N), 1)

    def valid_of(r):
        cidx = r // L
        rl = r % L
        sh = jnp.zeros_like(r)
        cl = jnp.zeros_like(r)
        for j in range(CB):
            sh = jnp.where(cidx == j, shifts[j], sh)
            cl = jnp.where(cidx == j, clens[j], cl)
        return (rl >= sh) & (rl < sh + cl)

    vcol = valid_of(rcol)
    vrow = valid_of(rrow)
    maskf = vcol.astype(jnp.float32)
    keybias = jnp.where(vrow, 0.0, NEG)

    for cp in copies:
        cp.wait()

    xb = xbuf[slot]
    x = xb.reshape(N, SRC) @ we[...] + be[...]
    x = x * maskf

    q = x @ wq[...]; k = x @ wk[...]; v = x @ wv[...]
    sbias = bias[...] + keybias
    os_ = []
    for h in range(H):
        s = lax.dot_general(q[:, DH*h:DH*(h+1)], k[:, DH*h:DH*(h+1)],
                            (((1,), (1,)), ((), ()))) * SCALE + sbias
        m = jnp.max(s, axis=1, keepdims=True)
        p = jnp.exp(s - m)
        os_.append((p @ v[:, DH*h:DH*(h+1)]) /
                   jnp.sum(p, axis=1, keepdims=True))
    o = jnp.concatenate(os_, axis=1) @ wo[...]
    h1 = x + o * maskf
    h2 = h1 + (jax.nn.gelu(h1 @ w1[...] + b1[...]) @ w2[...] + b2[...]) * maskf

    k2 = h2 @ wk2[...]; v2 = h2 @ wv2[...]
    qpe = (qc[...] + pe[...]).reshape(CB * Q, D)
    qg = qpe @ wq2[...]
    ri2 = lax.broadcasted_iota(jnp.int32, (CB * Q, N), 0) // Q
    ci2 = lax.broadcasted_iota(jnp.int32, (CB * Q, N), 1) // L
    allow2 = (ri2 == ci2) & vrow
    allow2f = allow2.astype(jnp.float32)
    gs = [_masked_attn(qg[:, DH*h:DH*(h+1)], k2[:, DH*h:DH*(h+1)],
                       v2[:, DH*h:DH*(h+1)], allow2, allow2f) for h in range(H)]
    g = jnp.concatenate(gs, axis=1) @ wo2[...]
    out[...] = (qpe + pe[...].reshape(CB * Q, D) + g).reshape(CB, Q, D)


def _kv_kernel(tg_ref, wkg, wvg, kk_ref, vv_ref):
    t = tg_ref[...]
    kk_ref[...] = _mm(t, wkg[...])
    vv_ref[...] = _mm(t, wvg[...])


def _global_kernel(tg_ref, kk_ref, vv_ref, wqg, wog, wg1, bg1, wg2, bg2, z_ref):
    t = tg_ref[...]
    qq = _mm(t, wqg[...])
    kk = kk_ref[...]
    vv = vv_ref[...]
    outs = []
    for h in range(H):
        s = _mmt(qq[:, DH*h:DH*(h+1)], kk[:, DH*h:DH*(h+1)]) * SCALE
        m = jnp.max(s, axis=1, keepdims=True)
        p = jnp.exp(s - m)
        a = p / jnp.sum(p, axis=1, keepdims=True)
        outs.append(_mm(a, vv[:, DH*h:DH*(h+1)]))
    z = t + _mm(jnp.concatenate(outs, axis=1), wog[...])
    z = z + _mm(jax.nn.gelu(_mm(z, wg1[...]) + bg1[...]), wg2[...]) + bg2[...]
    z_ref[...] = z


def _full(shape):
    return pl.BlockSpec(shape, lambda i, *_: (0,) * len(shape))


def kernel(streams_data, segment_ids, pe_global, q_cells, W_embed, b_embed,
           Wq, Wk, Wv, Wo, W1, b1, W2, b2, Wq2, Wk2, Wv2, Wo2,
           Wqg, Wkg, Wvg, Wog, Wg1, bg1, Wg2, bg2):
    starts2, lens2 = pl.pallas_call(
        _seg_kernel,
        grid=(NSEG,),
        in_specs=[pl.BlockSpec((1, TB, 1), lambda i: (i, 0, 0))],
        out_specs=[pl.BlockSpec((1, C), lambda i: (0, 0))] * 2,
        out_shape=[jax.ShapeDtypeStruct((1, C), jnp.int32)] * 2,
        scratch_shapes=[pltpu.VMEM((1, C), jnp.float32)] * 2,
    )(segment_ids.reshape(NSEG, TB, 1))
    starts = starts2.reshape(C)
    lens = lens2.reshape(C)

    grid_spec = pltpu.PrefetchScalarGridSpec(
        num_scalar_prefetch=2,
        grid=(C // CB,),
        in_specs=[
            pl.BlockSpec(memory_space=pl.ANY),
            pl.BlockSpec((CB, Q, D), lambda i, *_: (i, 0, 0)),
            pl.BlockSpec((CB, Q, D), lambda i, *_: (i, 0, 0)),
            _full((N, N)),
            _full((SRC, D)), _full((1, D)),
            _full((D, D)), _full((D, D)), _full((D, D)), _full((D, D)),
            _full((D, FF)), _full((1, FF)), _full((FF, D)), _full((1, D)),
            _full((D, D)), _full((D, D)), _full((D, D)), _full((D, D)),
        ],
        out_specs=pl.BlockSpec((CB, Q, D), lambda i, *_: (i, 0, 0)),
        scratch_shapes=[pltpu.VMEM((2, CB, L, SRC), jnp.float32),
                        pltpu.SemaphoreType.DMA((2, CB))],
    )
    ii = np.arange(N) // L
    bias = jnp.asarray(np.where(ii[:, None] == ii[None, :], 0.0, NEG),
                       dtype=jnp.float32)
    tg = pl.pallas_call(
        _local_kernel,
        grid_spec=grid_spec,
        out_shape=jax.ShapeDtypeStruct((C, Q, D), jnp.float32),
    )(starts, lens, streams_data, q_cells, pe_global, bias,
      W_embed, b_embed.reshape(1, D), Wq, Wk, Wv, Wo,
      W1, b1.reshape(1, FF), W2, b2.reshape(1, D), Wq2, Wk2, Wv2, Wo2)

    tg2 = tg.reshape(S, D)
    kk, vv = pl.pallas_call(
        _kv_kernel,
        grid=(S // QB,),
        in_specs=[pl.BlockSpec((QB, D), lambda i: (i, 0)),
                  pl.BlockSpec((D, D), lambda i: (0, 0)),
                  pl.BlockSpec((D, D), lambda i: (0, 0))],
        out_specs=[pl.BlockSpec((QB, D), lambda i: (i, 0))] * 2,
        out_shape=[jax.ShapeDtypeStruct((S, D), jnp.float32)] * 2,
    )(tg2, Wkg, Wvg)

    z = pl.pallas_call(
        _global_kernel,
        grid=(S // QB,),
        in_specs=[pl.BlockSpec((QB, D), lambda i: (i, 0)),
                  pl.BlockSpec((S, D), lambda i: (0, 0)),
                  pl.BlockSpec((S, D), lambda i: (0, 0)),
                  pl.BlockSpec((D, D), lambda i: (0, 0)),
                  pl.BlockSpec((D, D), lambda i: (0, 0)),
                  pl.BlockSpec((D, FF), lambda i: (0, 0)),
                  pl.BlockSpec((1, FF), lambda i: (0, 0)),
                  pl.BlockSpec((FF, D), lambda i: (0, 0)),
                  pl.BlockSpec((1, D), lambda i: (0, 0))],
        out_specs=pl.BlockSpec((QB, D), lambda i: (i, 0)),
        out_shape=jax.ShapeDtypeStruct((S, D), jnp.float32),
    )(tg2, kk, vv, Wqg, Wog, Wg1, bg1.reshape(1, FF), Wg2, bg2.reshape(1, D))
    return z


# speedup vs baseline: 2.2234x; 1.7673x over previous
import jax
import jax.numpy as jnp
import numpy as np
from jax import lax
from jax.experimental import pallas as pl
from jax.experimental.pallas import tpu as pltpu

C = 768; T = 24576; SRC = 128; D = 256; L = 96; H = 4; DH = 64; Q = 4; FF = 1024
S = C * Q
NSEG = 12
TB = T // NSEG
BT = 512
HALO = 128
W = BT + 2 * HALO
NB = T // BT
CC = 8
QB = 768
WL = L + 8
SCALE = 0.125
NEG = -1e9


def _seg_kernel(seg_ref, starts_ref, lens_ref, acc_lt, acc_eq):
    i = pl.program_id(0)

    @pl.when(i == 0)
    def _init():
        acc_lt[...] = jnp.zeros_like(acc_lt)
        acc_eq[...] = jnp.zeros_like(acc_eq)

    seg = seg_ref[0]
    cells = lax.broadcasted_iota(jnp.int32, (1, C), 1)
    lt = (seg < cells).astype(jnp.float32)
    eq = (seg == cells).astype(jnp.float32)
    acc_lt[...] += jnp.sum(lt, axis=0, keepdims=True)
    acc_eq[...] += jnp.sum(eq, axis=0, keepdims=True)

    @pl.when(i == NSEG - 1)
    def _fin():
        starts_ref[...] = acc_lt[...].astype(jnp.int32)
        lens_ref[...] = acc_eq[...].astype(jnp.int32)


def _masked_attn(qh, kh, vh, allow, allowf):
    s = lax.dot_general(qh, kh, (((1,), (1,)), ((), ()))) * SCALE
    s = jnp.where(allow, s, NEG)
    m = jnp.max(s, axis=1, keepdims=True)
    p = jnp.exp(s - m) * allowf
    den = jnp.maximum(jnp.sum(p, axis=1, keepdims=True), 1e-30)
    return (p @ vh) / den


def _win_start(i):
    return pl.multiple_of(jnp.minimum(jnp.maximum(i * BT - HALO, 0), T - W),
                          HALO)


def _band_copies(streams, segr, segsh, sbuf, gbuf, hbuf, sems, step, slot):
    s0 = _win_start(step)
    return [
        pltpu.make_async_copy(streams.at[pl.ds(s0, W), :], sbuf.at[slot],
                              sems.at[slot, 0]),
        pltpu.make_async_copy(segr.at[:, pl.ds(s0, W)], gbuf.at[slot],
                              sems.at[slot, 1]),
        pltpu.make_async_copy(segsh.at[:, pl.ds(s0, W)], hbuf.at[slot],
                              sems.at[slot, 2]),
    ]


def _band_kernel(streams, segr, segsh, segq_ref, we, be, wq, wk, wv, wo,
                 w1, b1, w2, b2, wk2, wv2, k2_ref, v2_ref,
                 sbuf, gbuf, hbuf, sems):
    pid = pl.program_id(0)
    G = pl.num_programs(0)
    slot = lax.rem(pid, 2)

    @pl.when(pid == 0)
    def _prime():
        for cp in _band_copies(streams, segr, segsh, sbuf, gbuf, hbuf, sems,
                               0, 0):
            cp.start()

    @pl.when(pid < G - 1)
    def _next():
        for cp in _band_copies(streams, segr, segsh, sbuf, gbuf, hbuf, sems,
                               pid + 1, lax.rem(pid + 1, 2)):
            cp.start()

    for cp in _band_copies(streams, segr, segsh, sbuf, gbuf, hbuf, sems,
                           pid, slot):
        cp.wait()

    s0 = _win_start(pid)
    qoff = pl.multiple_of(pid * BT - s0, HALO)

    xw = sbuf[slot] @ we[...] + be[...]
    xq = sbuf[slot, pl.ds(qoff, BT), :] @ we[...] + be[...]
    k = xw @ wk[...]
    v = xw @ wv[...]
    q = xq @ wq[...]

    segk = gbuf[slot]
    keep_k = segk != hbuf[slot]
    segq = segq_ref[...]
    allow = (segq == segk) & keep_k
    sbias = jnp.where(allow, 0.0, NEG)

    os_ = []
    for h in range(H):
        s = lax.dot_general(q[:, DH*h:DH*(h+1)], k[:, DH*h:DH*(h+1)],
                            (((1,), (1,)), ((), ()))) * SCALE + sbias
        m = jnp.max(s, axis=1, keepdims=True)
        p = jnp.exp(s - m)
        os_.append((p @ v[:, DH*h:DH*(h+1)]) /
                   jnp.sum(p, axis=1, keepdims=True))
    o = jnp.concatenate(os_, axis=1) @ wo[...]
    h1 = xq + o
    h2 = h1 + jax.nn.gelu(h1 @ w1[...] + b1[...]) @ w2[...] + b2[...]
    k2_ref[...] = h2 @ wk2[...]
    v2_ref[...] = h2 @ wv2[...]


def _cross_copies(starts, lens, k2a, v2a, kbuf, vbuf, semk, semv, step, slot):
    c0 = step * CC
    shifts, clens, copies = [], [], []
    for j in range(CC):
        st = starts[c0 + j]
        cl = jnp.minimum(lens[c0 + j], L)
        s0 = pl.multiple_of((jnp.minimum(st, T - WL) // 8) * 8, 8)
        shifts.append(st - s0)
        clens.append(cl)
        copies.append(pltpu.make_async_copy(
            k2a.at[pl.ds(s0, WL), :], kbuf.at[slot, j], semk.at[slot, j]))
        copies.append(pltpu.make_async_copy(
            v2a.at[pl.ds(s0, WL), :], vbuf.at[slot, j], semv.at[slot, j]))
    return shifts, clens, copies


def _cross_kernel(starts, lens, k2a, v2a, qc, pe, wq2, wo2, out,
                  kbuf, vbuf, semk, semv):
    pid = pl.program_id(0)
    G = pl.num_programs(0)
    slot = lax.rem(pid, 2)

    @pl.when(pid == 0)
    def _prime():
        for cp in _cross_copies(starts, lens, k2a, v2a, kbuf, vbuf,
                                semk, semv, 0, 0)[2]:
            cp.start()

    @pl.when(pid < G - 1)
    def _next():
        for cp in _cross_copies(starts, lens, k2a, v2a, kbuf, vbuf,
                                semk, semv, pid + 1, lax.rem(pid + 1, 2))[2]:
            cp.start()

    shifts, clens, copies = _cross_copies(starts, lens, k2a, v2a, kbuf, vbuf,
                                          semk, semv, pid, slot)

    rrow = lax.broadcasted_iota(jnp.int32, (1, CC * WL), 1)
    cidx = rrow // WL
    rl = rrow % WL
    sh = jnp.zeros_like(rrow)
    cl = jnp.zeros_like(rrow)
    for j in range(CC):
        sh = jnp.where(cidx == j, shifts[j], sh)
        cl = jnp.where(cidx == j, clens[j], cl)
    vrow = (rl >= sh) & (rl < sh + cl)

    for cp in copies:
        cp.wait()

    k2 = kbuf[slot].reshape(CC * WL, D)
    v2 = vbuf[slot].reshape(CC * WL, D)
    qpe = (qc[...] + pe[...]).reshape(CC * Q, D)
    qg = qpe @ wq2[...]
    ri2 = lax.broadcasted_iota(jnp.int32, (CC * Q, CC * WL), 0) // Q
    ci2 = lax.broadcasted_iota(jnp.int32, (CC * Q, CC * WL), 1) // WL
    allow2 = (ri2 == ci2) & vrow
    allow2f = allow2.astype(jnp.float32)
    gs = [_masked_attn(qg[:, DH*h:DH*(h+1)], k2[:, DH*h:DH*(h+1)],
                       v2[:, DH*h:DH*(h+1)], allow2, allow2f) for h in range(H)]
    g = jnp.concatenate(gs, axis=1) @ wo2[...]
    out[...] = (qpe + pe[...].reshape(CC * Q, D) + g).reshape(CC, Q, D)


def _kv_kernel(tg_ref, wkg, wvg, kk_ref, vv_ref):
    t = tg_ref[...]
    kk_ref[...] = t @ wkg[...]
    vv_ref[...] = t @ wvg[...]


def _global_kernel(tg_ref, kk_ref, vv_ref, wqg, wog, wg1, bg1, wg2, bg2, z_ref):
    t = tg_ref[...]
    qq = t @ wqg[...]
    kk = kk_ref[...]
    vv = vv_ref[...]
    outs = []
    for h in range(H):
        s = lax.dot_general(qq[:, DH*h:DH*(h+1)], kk[:, DH*h:DH*(h+1)],
                            (((1,), (1,)), ((), ()))) * SCALE
        m = jnp.max(s, axis=1, keepdims=True)
        p = jnp.exp(s - m)
        a = p / jnp.sum(p, axis=1, keepdims=True)
        outs.append(a @ vv[:, DH*h:DH*(h+1)])
    z = t + jnp.concatenate(outs, axis=1) @ wog[...]
    z = z + jax.nn.gelu(z @ wg1[...] + bg1[...]) @ wg2[...] + bg2[...]
    z_ref[...] = z


def _full(shape):
    return pl.BlockSpec(shape, lambda i, *_: (0,) * len(shape))


def kernel(streams_data, segment_ids, pe_global, q_cells, W_embed, b_embed,
           Wq, Wk, Wv, Wo, W1, b1, W2, b2, Wq2, Wk2, Wv2, Wo2,
           Wqg, Wkg, Wvg, Wog, Wg1, bg1, Wg2, bg2):
    seg = segment_ids.astype(jnp.int32)
    starts2, lens2 = pl.pallas_call(
        _seg_kernel,
        grid=(NSEG,),
        in_specs=[pl.BlockSpec((1, TB, 1), lambda i: (i, 0, 0))],
        out_specs=[pl.BlockSpec((1, C), lambda i: (0, 0))] * 2,
        out_shape=[jax.ShapeDtypeStruct((1, C), jnp.int32)] * 2,
        scratch_shapes=[pltpu.VMEM((1, C), jnp.float32)] * 2,
    )(seg.reshape(NSEG, TB, 1))
    starts = starts2.reshape(C)
    lens = lens2.reshape(C)

    seg_row = seg.reshape(1, T)
    seg_sh_row = jnp.concatenate(
        [jnp.full((1, L), -1, jnp.int32), seg_row[:, :-L]], axis=1)

    k2p, v2p = pl.pallas_call(
        _band_kernel,
        grid=(NB,),
        in_specs=[
            pl.BlockSpec(memory_space=pl.ANY),
            pl.BlockSpec(memory_space=pl.ANY),
            pl.BlockSpec(memory_space=pl.ANY),
            pl.BlockSpec((BT, 1), lambda i: (i, 0)),
            _full((SRC, D)), _full((1, D)),
            _full((D, D)), _full((D, D)), _full((D, D)), _full((D, D)),
            _full((D, FF)), _full((1, FF)), _full((FF, D)), _full((1, D)),
            _full((D, D)), _full((D, D)),
        ],
        out_specs=[pl.BlockSpec((BT, D), lambda i: (i, 0))] * 2,
        out_shape=[jax.ShapeDtypeStruct((T, D), jnp.float32)] * 2,
        scratch_shapes=[pltpu.VMEM((2, W, SRC), jnp.float32),
                        pltpu.VMEM((2, 1, W), jnp.int32),
                        pltpu.VMEM((2, 1, W), jnp.int32),
                        pltpu.SemaphoreType.DMA((2, 3))],
    )(streams_data, seg_row, seg_sh_row, seg.reshape(T, 1),
      W_embed, b_embed.reshape(1, D), Wq, Wk, Wv, Wo,
      W1, b1.reshape(1, FF), W2, b2.reshape(1, D), Wk2, Wv2)

    grid_spec = pltpu.PrefetchScalarGridSpec(
        num_scalar_prefetch=2,
        grid=(C // CC,),
        in_specs=[
            pl.BlockSpec(memory_space=pl.ANY),
            pl.BlockSpec(memory_space=pl.ANY),
            pl.BlockSpec((CC, Q, D), lambda i, *_: (i, 0, 0)),
            pl.BlockSpec((CC, Q, D), lambda i, *_: (i, 0, 0)),
            _full((D, D)), _full((D, D)),
        ],
        out_specs=pl.BlockSpec((CC, Q, D), lambda i, *_: (i, 0, 0)),
        scratch_shapes=[pltpu.VMEM((2, CC, WL, D), jnp.float32),
                        pltpu.VMEM((2, CC, WL, D), jnp.float32),
                        pltpu.SemaphoreType.DMA((2, CC)),
                        pltpu.SemaphoreType.DMA((2, CC))],
    )
    tg = pl.pallas_call(
        _cross_kernel,
        grid_spec=grid_spec,
        out_shape=jax.ShapeDtypeStruct((C, Q, D), jnp.float32),
    )(starts, lens, k2p, v2p, q_cells, pe_global, Wq2, Wo2)

    tg2 = tg.reshape(S, D)
    kk, vv = pl.pallas_call(
        _kv_kernel,
        grid=(S // QB,),
        in_specs=[pl.BlockSpec((QB, D), lambda i: (i, 0)),
                  pl.BlockSpec((D, D), lambda i: (0, 0)),
                  pl.BlockSpec((D, D), lambda i: (0, 0))],
        out_specs=[pl.BlockSpec((QB, D), lambda i: (i, 0))] * 2,
        out_shape=[jax.ShapeDtypeStruct((S, D), jnp.float32)] * 2,
    )(tg2, Wkg, Wvg)

    z = pl.pallas_call(
        _global_kernel,
        grid=(S // QB,),
        in_specs=[pl.BlockSpec((QB, D), lambda i: (i, 0)),
                  pl.BlockSpec((S, D), lambda i: (0, 0)),
                  pl.BlockSpec((S, D), lambda i: (0, 0)),
                  pl.BlockSpec((D, D), lambda i: (0, 0)),
                  pl.BlockSpec((D, D), lambda i: (0, 0)),
                  pl.BlockSpec((D, FF), lambda i: (0, 0)),
                  pl.BlockSpec((1, FF), lambda i: (0, 0)),
                  pl.BlockSpec((FF, D), lambda i: (0, 0)),
                  pl.BlockSpec((1, D), lambda i: (0, 0))],
        out_specs=pl.BlockSpec((QB, D), lambda i: (i, 0)),
        out_shape=jax.ShapeDtypeStruct((S, D), jnp.float32),
    )(tg2, kk, vv, Wqg, Wog, Wg1, bg1.reshape(1, FF), Wg2, bg2.reshape(1, D))
    return z


# speedup vs baseline: 2.2358x; 1.0056x over previous
import jax
import jax.numpy as jnp
import numpy as np
from jax import lax
from jax.experimental import pallas as pl
from jax.experimental.pallas import tpu as pltpu

C = 768; T = 24576; SRC = 128; D = 256; L = 96; H = 4; DH = 64; Q = 4; FF = 1024
S = C * Q
NSEG = 12
TB = T // NSEG
BT = 512
HALO = 128
W = BT + 2 * HALO
NB = T // BT
CC = 8
QB = 768
WL = L + 8
SCALE = 0.125
NEG = -1e9


def _seg_kernel(seg_ref, starts_ref, lens_ref, acc_lt, acc_eq):
    i = pl.program_id(0)

    @pl.when(i == 0)
    def _init():
        acc_lt[...] = jnp.zeros_like(acc_lt)
        acc_eq[...] = jnp.zeros_like(acc_eq)

    seg = seg_ref[0]
    cells = lax.broadcasted_iota(jnp.int32, (1, C), 1)
    lt = (seg < cells).astype(jnp.float32)
    eq = (seg == cells).astype(jnp.float32)
    acc_lt[...] += jnp.sum(lt, axis=0, keepdims=True)
    acc_eq[...] += jnp.sum(eq, axis=0, keepdims=True)

    @pl.when(i == NSEG - 1)
    def _fin():
        starts_ref[...] = acc_lt[...].astype(jnp.int32)
        lens_ref[...] = acc_eq[...].astype(jnp.int32)


def _masked_attn(qh, kh, vh, allow, allowf):
    s = lax.dot_general(qh, kh, (((1,), (1,)), ((), ()))) * SCALE
    s = jnp.where(allow, s, NEG)
    m = jnp.max(s, axis=1, keepdims=True)
    p = jnp.exp(s - m) * allowf
    den = jnp.maximum(jnp.sum(p, axis=1, keepdims=True), 1e-30)
    return (p @ vh) / den


def _win_start(i):
    return pl.multiple_of(jnp.minimum(jnp.maximum(i * BT - HALO, 0), T - W),
                          HALO)


def _band_copies(streams, segr, segsh, sbuf, gbuf, hbuf, sems, step, slot):
    s0 = _win_start(step)
    return [
        pltpu.make_async_copy(streams.at[pl.ds(s0, W), :], sbuf.at[slot],
                              sems.at[slot, 0]),
        pltpu.make_async_copy(segr.at[:, pl.ds(s0, W)], gbuf.at[slot],
                              sems.at[slot, 1]),
        pltpu.make_async_copy(segsh.at[:, pl.ds(s0, W)], hbuf.at[slot],
                              sems.at[slot, 2]),
    ]


def _band_kernel(streams, segr, segsh, segq_ref, we, be, wq, wk, wv, wo,
                 w1, b1, w2, b2, wk2, wv2, kv2_ref,
                 sbuf, gbuf, hbuf, sems):
    pid = pl.program_id(0)
    G = pl.num_programs(0)
    slot = lax.rem(pid, 2)

    @pl.when(pid == 0)
    def _prime():
        for cp in _band_copies(streams, segr, segsh, sbuf, gbuf, hbuf, sems,
                               0, 0):
            cp.start()

    @pl.when(pid < G - 1)
    def _next():
        for cp in _band_copies(streams, segr, segsh, sbuf, gbuf, hbuf, sems,
                               pid + 1, lax.rem(pid + 1, 2)):
            cp.start()

    for cp in _band_copies(streams, segr, segsh, sbuf, gbuf, hbuf, sems,
                           pid, slot):
        cp.wait()

    s0 = _win_start(pid)
    qoff = pl.multiple_of(pid * BT - s0, HALO)

    xw = sbuf[slot] @ we[...] + be[...]
    xq = sbuf[slot, pl.ds(qoff, BT), :] @ we[...] + be[...]
    k = xw @ wk[...]
    v = xw @ wv[...]
    q = xq @ wq[...]

    segk = gbuf[slot]
    keep_k = segk != hbuf[slot]
    segq = segq_ref[...]
    allow = (segq == segk) & keep_k
    sbias = jnp.where(allow, 0.0, NEG)

    os_ = []
    for h in range(H):
        s = lax.dot_general(q[:, DH*h:DH*(h+1)], k[:, DH*h:DH*(h+1)],
                            (((1,), (1,)), ((), ()))) * SCALE + sbias
        m = jnp.max(s, axis=1, keepdims=True)
        p = jnp.exp(s - m)
        os_.append((p @ v[:, DH*h:DH*(h+1)]) /
                   jnp.sum(p, axis=1, keepdims=True))
    o = jnp.concatenate(os_, axis=1) @ wo[...]
    h1 = xq + o
    h2 = h1 + jax.nn.gelu(h1 @ w1[...] + b1[...]) @ w2[...] + b2[...]
    kv2_ref[...] = jnp.concatenate(
        [h2 @ wk2[...], h2 @ wv2[...]], axis=1).astype(jnp.bfloat16)


def _cross_copies(starts, lens, kv2a, kvbuf, semkv, step, slot):
    c0 = step * CC
    shifts, clens, copies = [], [], []
    for j in range(CC):
        st = starts[c0 + j]
        cl = jnp.minimum(lens[c0 + j], L)
        s0 = pl.multiple_of((jnp.minimum(st, T - WL) // 8) * 8, 8)
        shifts.append(st - s0)
        clens.append(cl)
        copies.append(pltpu.make_async_copy(
            kv2a.at[pl.ds(s0, WL), :], kvbuf.at[slot, j], semkv.at[slot, j]))
    return shifts, clens, copies


def _cross_kernel(starts, lens, kv2a, qc, pe, wq2, wo2, out,
                  kvbuf, semkv):
    pid = pl.program_id(0)
    G = pl.num_programs(0)
    slot = lax.rem(pid, 2)

    @pl.when(pid == 0)
    def _prime():
        for cp in _cross_copies(starts, lens, kv2a, kvbuf, semkv, 0, 0)[2]:
            cp.start()

    @pl.when(pid < G - 1)
    def _next():
        for cp in _cross_copies(starts, lens, kv2a, kvbuf, semkv,
                                pid + 1, lax.rem(pid + 1, 2))[2]:
            cp.start()

    shifts, clens, copies = _cross_copies(starts, lens, kv2a, kvbuf, semkv,
                                          pid, slot)

    rrow = lax.broadcasted_iota(jnp.int32, (1, CC * WL), 1)
    cidx = rrow // WL
    rl = rrow % WL
    sh = jnp.zeros_like(rrow)
    cl = jnp.zeros_like(rrow)
    for j in range(CC):
        sh = jnp.where(cidx == j, shifts[j], sh)
        cl = jnp.where(cidx == j, clens[j], cl)
    vrow = (rl >= sh) & (rl < sh + cl)

    for cp in copies:
        cp.wait()

    kv2 = kvbuf[slot].reshape(CC * WL, 2 * D)
    k2 = kv2[:, :D].astype(jnp.float32)
    v2 = kv2[:, D:].astype(jnp.float32)
    qpe = (qc[...] + pe[...]).reshape(CC * Q, D)
    qg = qpe @ wq2[...]
    ri2 = lax.broadcasted_iota(jnp.int32, (CC * Q, CC * WL), 0) // Q
    ci2 = lax.broadcasted_iota(jnp.int32, (CC * Q, CC * WL), 1) // WL
    allow2 = (ri2 == ci2) & vrow
    allow2f = allow2.astype(jnp.float32)
    gs = [_masked_attn(qg[:, DH*h:DH*(h+1)], k2[:, DH*h:DH*(h+1)],
                       v2[:, DH*h:DH*(h+1)], allow2, allow2f) for h in range(H)]
    g = jnp.concatenate(gs, axis=1) @ wo2[...]
    out[...] = (qpe + pe[...].reshape(CC * Q, D) + g).reshape(CC, Q, D)


def _kv_kernel(tg_ref, wkg, wvg, kk_ref, vv_ref):
    t = tg_ref[...]
    kk_ref[...] = t @ wkg[...]
    vv_ref[...] = t @ wvg[...]


def _global_kernel(tg_ref, kk_ref, vv_ref, wqg, wog, wg1, bg1, wg2, bg2, z_ref):
    t = tg_ref[...]
    qq = t @ wqg[...]
    kk = kk_ref[...]
    vv = vv_ref[...]
    outs = []
    for h in range(H):
        s = lax.dot_general(qq[:, DH*h:DH*(h+1)], kk[:, DH*h:DH*(h+1)],
                            (((1,), (1,)), ((), ()))) * SCALE
        m = jnp.max(s, axis=1, keepdims=True)
        p = jnp.exp(s - m)
        a = p / jnp.sum(p, axis=1, keepdims=True)
        outs.append(a @ vv[:, DH*h:DH*(h+1)])
    z = t + jnp.concatenate(outs, axis=1) @ wog[...]
    z = z + jax.nn.gelu(z @ wg1[...] + bg1[...]) @ wg2[...] + bg2[...]
    z_ref[...] = z


def _full(shape):
    return pl.BlockSpec(shape, lambda i, *_: (0,) * len(shape))


def kernel(streams_data, segment_ids, pe_global, q_cells, W_embed, b_embed,
           Wq, Wk, Wv, Wo, W1, b1, W2, b2, Wq2, Wk2, Wv2, Wo2,
           Wqg, Wkg, Wvg, Wog, Wg1, bg1, Wg2, bg2):
    seg = segment_ids.astype(jnp.int32)
    starts2, lens2 = pl.pallas_call(
        _seg_kernel,
        grid=(NSEG,),
        in_specs=[pl.BlockSpec((1, TB, 1), lambda i: (i, 0, 0))],
        out_specs=[pl.BlockSpec((1, C), lambda i: (0, 0))] * 2,
        out_shape=[jax.ShapeDtypeStruct((1, C), jnp.int32)] * 2,
        scratch_shapes=[pltpu.VMEM((1, C), jnp.float32)] * 2,
    )(seg.reshape(NSEG, TB, 1))
    starts = starts2.reshape(C)
    lens = lens2.reshape(C)

    seg_row = seg.reshape(1, T)
    seg_sh_row = jnp.concatenate(
        [jnp.full((1, L), -1, jnp.int32), seg_row[:, :-L]], axis=1)

    kv2p = pl.pallas_call(
        _band_kernel,
        grid=(NB,),
        in_specs=[
            pl.BlockSpec(memory_space=pl.ANY),
            pl.BlockSpec(memory_space=pl.ANY),
            pl.BlockSpec(memory_space=pl.ANY),
            pl.BlockSpec((BT, 1), lambda i: (i, 0)),
            _full((SRC, D)), _full((1, D)),
            _full((D, D)), _full((D, D)), _full((D, D)), _full((D, D)),
            _full((D, FF)), _full((1, FF)), _full((FF, D)), _full((1, D)),
            _full((D, D)), _full((D, D)),
        ],
        out_specs=pl.BlockSpec((BT, 2 * D), lambda i: (i, 0)),
        out_shape=jax.ShapeDtypeStruct((T, 2 * D), jnp.bfloat16),
        scratch_shapes=[pltpu.VMEM((2, W, SRC), jnp.float32),
                        pltpu.VMEM((2, 1, W), jnp.int32),
                        pltpu.VMEM((2, 1, W), jnp.int32),
                        pltpu.SemaphoreType.DMA((2, 3))],
    )(streams_data, seg_row, seg_sh_row, seg.reshape(T, 1),
      W_embed, b_embed.reshape(1, D), Wq, Wk, Wv, Wo,
      W1, b1.reshape(1, FF), W2, b2.reshape(1, D), Wk2, Wv2)

    grid_spec = pltpu.PrefetchScalarGridSpec(
        num_scalar_prefetch=2,
        grid=(C // CC,),
        in_specs=[
            pl.BlockSpec(memory_space=pl.ANY),
            pl.BlockSpec((CC, Q, D), lambda i, *_: (i, 0, 0)),
            pl.BlockSpec((CC, Q, D), lambda i, *_: (i, 0, 0)),
            _full((D, D)), _full((D, D)),
        ],
        out_specs=pl.BlockSpec((CC, Q, D), lambda i, *_: (i, 0, 0)),
        scratch_shapes=[pltpu.VMEM((2, CC, WL, 2 * D), jnp.bfloat16),
                        pltpu.SemaphoreType.DMA((2, CC))],
    )
    tg = pl.pallas_call(
        _cross_kernel,
        grid_spec=grid_spec,
        out_shape=jax.ShapeDtypeStruct((C, Q, D), jnp.float32),
    )(starts, lens, kv2p, q_cells, pe_global, Wq2, Wo2)

    tg2 = tg.reshape(S, D)
    kk, vv = pl.pallas_call(
        _kv_kernel,
        grid=(S // QB,),
        in_specs=[pl.BlockSpec((QB, D), lambda i: (i, 0)),
                  pl.BlockSpec((D, D), lambda i: (0, 0)),
                  pl.BlockSpec((D, D), lambda i: (0, 0))],
        out_specs=[pl.BlockSpec((QB, D), lambda i: (i, 0))] * 2,
        out_shape=[jax.ShapeDtypeStruct((S, D), jnp.float32)] * 2,
    )(tg2, Wkg, Wvg)

    z = pl.pallas_call(
        _global_kernel,
        grid=(S // QB,),
        in_specs=[pl.BlockSpec((QB, D), lambda i: (i, 0)),
                  pl.BlockSpec((S, D), lambda i: (0, 0)),
                  pl.BlockSpec((S, D), lambda i: (0, 0)),
                  pl.BlockSpec((D, D), lambda i: (0, 0)),
                  pl.BlockSpec((D, D), lambda i: (0, 0)),
                  pl.BlockSpec((D, FF), lambda i: (0, 0)),
                  pl.BlockSpec((1, FF), lambda i: (0, 0)),
                  pl.BlockSpec((FF, D), lambda i: (0, 0)),
                  pl.BlockSpec((1, D), lambda i: (0, 0))],
        out_specs=pl.BlockSpec((QB, D), lambda i: (i, 0)),
        out_shape=jax.ShapeDtypeStruct((S, D), jnp.float32),
    )(tg2, kk, vv, Wqg, Wog, Wg1, bg1.reshape(1, FF), Wg2, bg2.reshape(1, D))
    return z
